# baseline (speedup 1.0000x reference)
import functools

import jax
import jax.numpy as jnp
from jax import lax
from jax.experimental import pallas as pl
from jax.experimental.pallas import tpu as pltpu
from jax.experimental.pallas import tpu_sc as plsc

NC = 2
NS = 16
SU = 40
CHUNK = 7936


@jax.jit
def _sc_embedding_lookup(ids2, w_t, w_tail):
    D, V = w_t.shape
    S, B = 200, 4096
    b_per_t = B // (NC * NS)
    unit = SU * b_per_t
    n_u = S // SU
    va = (V // 128) * 128
    tail = V - va
    n_chunks = va // CHUNK
    mesh = plsc.VectorSubcoreMesh(core_axis_name="c", subcore_axis_name="s")

    @functools.partial(
        pl.kernel,
        out_type=jax.ShapeDtypeStruct((S, D // 8, 8, B), jnp.float32),
        mesh=mesh,
        scratch_types=[
            pltpu.VMEM_SHARED((V,), jnp.float32),
            pltpu.VMEM((S * b_per_t,), jnp.int32),
            pltpu.VMEM((CHUNK,), jnp.float32),
            [pltpu.VMEM((1, unit), jnp.float32) for _ in range(2)],
            [pltpu.SemaphoreType.DMA for _ in range(2)],
            [pltpu.SemaphoreType.DMA for _ in range(2)],
        ],
    )
    def body(ids_hbm, w_hbm, wt_hbm, out_hbm, col_sh, idx_v, stg_v, dst,
             sem_g, sem_p):
        cid = lax.axis_index("c")
        sid = lax.axis_index("s")
        wid = cid * NS + sid
        b0 = wid * b_per_t
        pltpu.sync_copy(ids_hbm.at[wid], idx_v)

        def stage(c):
            for r in range((n_chunks + NS - 1) // NS):
                k = sid + r * NS

                @pl.when(k < n_chunks)
                def _():
                    pltpu.sync_copy(w_hbm.at[c, pl.ds(k * CHUNK, CHUNK)], stg_v)
                    pltpu.sync_copy(stg_v, col_sh.at[pl.ds(k * CHUNK, CHUNK)])

            @pl.when(sid == 0)
            def _():
                pltpu.sync_copy(wt_hbm.at[pl.ds(c * tail, tail)],
                                stg_v.at[pl.ds(0, tail)])
                pltpu.sync_copy(stg_v.at[pl.ds(0, tail)],
                                col_sh.at[pl.ds(va, tail)])

        def fire_gather(u, b):
            pltpu.async_copy(col_sh.at[idx_v.at[pl.ds(u * unit, unit)]],
                             dst[b].at[0], sem_g[b])

        def wait_gather(u, b):
            pltpu.make_async_copy(col_sh.at[idx_v.at[pl.ds(u * unit, unit)]],
                                  dst[b].at[0], sem_g[b]).wait()

        def fire_put(c, u, b):
            pltpu.async_copy(
                dst[b].reshape(SU, b_per_t),
                out_hbm.at[pl.ds(u * SU, SU), c // 8, c % 8, pl.ds(b0, b_per_t)],
                sem_p[b])

        def wait_put(b):
            pltpu.make_async_copy(
                dst[b].reshape(SU, b_per_t),
                out_hbm.at[pl.ds(0, SU), 0, 0, pl.ds(b0, b_per_t)],
                sem_p[b]).wait()

        stage(0)
        plsc.subcore_barrier()
        fire_gather(0, 0)

        def pair_step(i, carry):
            for jj in range(2):
                c = 2 * i + jj
                for u in range(n_u):
                    b = (jj * n_u + u) % 2
                    wait_gather(u, b)
                    fire_put(c, u, b)
                    if u < n_u - 1:
                        if jj == 0 and u == 0:
                            @pl.when(i > 0)
                            def _():
                                wait_put(1 - b)
                        else:
                            wait_put(1 - b)
                        fire_gather(u + 1, 1 - b)
                    else:
                        plsc.subcore_barrier()

                        @pl.when(c + 1 < D)
                        def _():
                            stage(c + 1)

                        plsc.subcore_barrier()

                        @pl.when(c + 1 < D)
                        def _():
                            wait_put(1 - b)
                            fire_gather(0, 1 - b)
            return carry

        lax.fori_loop(0, D // 2, pair_step, 0)
        wait_put(0)
        wait_put(1)

    return body(ids2, w_t, w_tail)


def kernel(input_ids, weight):
    ids_t = input_ids.T.astype(jnp.int32)
    w_t = weight.T
    S, B = ids_t.shape
    V = weight.shape[0]
    va = (V // 128) * 128
    nw = NC * NS
    bpt = B // nw
    ids2 = ids_t.reshape(S, nw, bpt).transpose(1, 0, 2).reshape(nw, S * bpt)
    w_tail = weight[va:, :].T.reshape(-1)
    out_p = _sc_embedding_lookup(ids2, w_t, w_tail)
    return out_p.reshape(S, -1, B).transpose(2, 0, 1)
